# Initial kernel scaffold; baseline (speedup 1.0000x reference)
import jax
import jax.numpy as jnp
from jax.experimental import pallas as pl


def kernel(x, edge_index, Ws, bs, gammas, betas):
    raise NotImplementedError("write your pallas kernel here")


# speedup vs baseline: 7.4036x; 7.4036x over previous
import functools

import jax
import jax.numpy as jnp
from jax import lax
from jax.experimental import pallas as pl
from jax.experimental.pallas import tpu as pltpu
from jax.experimental.pallas import tpu_sc as plsc

N = 10000
E = 320000
D = 128

NC = 2
NS = 16
NW = NC * NS
EPT = E // NW
CHUNK = 80
NCHUNK = EPT // CHUNK
INIT_R = 624

_mesh = plsc.VectorSubcoreMesh(core_axis_name="c", subcore_axis_name="s")


def _edge_agg_body(ht_hbm, src_hbm, dst_hbm, out_hbm, src_v, dst_v, rows_v, acc, sem):
    c = lax.axis_index("c")
    s = lax.axis_index("s")
    wid = s * NC + c

    @pl.when(s < NS - 1)
    def _():
        pltpu.sync_copy(ht_hbm.at[pl.ds(s * INIT_R, INIT_R)],
                        acc.at[pl.ds(s * INIT_R, INIT_R)])

    @pl.when(s == NS - 1)
    def _():
        pltpu.sync_copy(ht_hbm.at[pl.ds((NS - 1) * INIT_R, N - (NS - 1) * INIT_R)],
                        acc.at[pl.ds((NS - 1) * INIT_R, N - (NS - 1) * INIT_R)])

    plsc.subcore_barrier()

    def body(i, _):
        pltpu.sync_copy(src_hbm.at[wid, i], src_v.at[0])
        pltpu.sync_copy(dst_hbm.at[wid, i], dst_v.at[0])
        pltpu.async_copy(ht_hbm.at[src_v.at[0]], rows_v, sem).wait()
        pltpu.sync_copy(rows_v, acc.at[dst_v.at[0]], add=True)
        return 0

    lax.fori_loop(0, NCHUNK, body, 0)

    plsc.subcore_barrier()

    @pl.when(s == 0)
    def _():
        pltpu.sync_copy(acc, out_hbm.at[c])


def _make_edge_agg(interpret=False):
    return pl.kernel(
        _edge_agg_body,
        out_type=jax.ShapeDtypeStruct((NC, N, D), jnp.float32),
        mesh=_mesh,
        scratch_types=[
            pltpu.VMEM((1, CHUNK), jnp.int32),
            pltpu.VMEM((1, CHUNK), jnp.int32),
            pltpu.VMEM((CHUNK, D), jnp.float32),
            pltpu.VMEM_SHARED((N, D), jnp.float32),
            pltpu.SemaphoreType.DMA,
        ],
        interpret=interpret,
    )


_edge_agg = _make_edge_agg()


def _mm(a, b):
    return jnp.dot(a, b, precision=lax.Precision.HIGHEST,
                   preferred_element_type=jnp.float32)


def _tc_pre_body(x_ref, w_ref, cnt_ref, ht_ref, dinv_ref):
    deg = cnt_ref[0, :, 0:1] + cnt_ref[1, :, 0:1] - 1.0
    dinv = lax.rsqrt(deg)
    dinv_ref[...] = dinv
    ht_ref[...] = _mm(x_ref[...], w_ref[...]) * dinv


_tc_pre = pl.pallas_call(
    _tc_pre_body,
    out_shape=[
        jax.ShapeDtypeStruct((N, D), jnp.float32),
        jax.ShapeDtypeStruct((N, 1), jnp.float32),
    ],
)


def _tc_mid_body(seg_ref, ht_ref, dinv_ref, b_ref, g_ref, bt_ref, w_ref, out_ref):
    dinv = dinv_ref[...]
    y = (seg_ref[0] + seg_ref[1] - ht_ref[...]) * dinv + b_ref[...]
    mu = jnp.mean(y, axis=0, keepdims=True)
    var = jnp.mean(y * y, axis=0, keepdims=True) - mu * mu
    yn = (y - mu) * lax.rsqrt(var + 1e-5) * g_ref[...] + bt_ref[...]
    r = jnp.maximum(yn, 0.0)
    out_ref[...] = _mm(r, w_ref[...]) * dinv


_tc_mid = pl.pallas_call(
    _tc_mid_body,
    out_shape=jax.ShapeDtypeStruct((N, D), jnp.float32),
)


def _tc_fin_body(seg_ref, ht_ref, dinv_ref, b_ref, out_ref):
    out_ref[...] = (seg_ref[0] + seg_ref[1] - ht_ref[...]) * dinv_ref[...] + b_ref[...]


_tc_fin = pl.pallas_call(
    _tc_fin_body,
    out_shape=jax.ShapeDtypeStruct((N, D), jnp.float32),
)


def kernel(x, edge_index, Ws, bs, gammas, betas):
    src = edge_index[0].reshape(NW, NCHUNK, CHUNK)
    dst = edge_index[1].reshape(NW, NCHUNK, CHUNK)
    bs2 = bs.reshape(12, 1, D)
    gs2 = gammas.reshape(11, 1, D)
    bts2 = betas.reshape(11, 1, D)

    cnt = _edge_agg(jnp.ones((N, D), jnp.float32), src, dst)
    ht, dinv = _tc_pre(x, Ws[0], cnt)
    for i in range(11):
        seg = _edge_agg(ht, src, dst)
        ht = _tc_mid(seg, ht, dinv, bs2[i], gs2[i], bts2[i], Ws[i + 1])
    seg = _edge_agg(ht, src, dst)
    return _tc_fin(seg, ht, dinv, bs2[11])
